# baseline (speedup 1.0000x reference)
import jax
import jax.numpy as jnp
from jax import lax
from jax.experimental import pallas as pl
from jax.experimental.pallas import tpu as pltpu
from jax.experimental.pallas import tpu_sc as plsc

B = 16384
L = 200
D = 8
NW = 32
ROWS_PER_W = B // NW
CBL = 128
NCH = ROWS_PER_W // CBL
CH_IDX = CBL * L
SUBS = 20
SR = CH_IDX // SUBS
NSTEP = NCH * SUBS

FMT_GRID = 16
FMT_COLS = 65536
VP = FMT_GRID * FMT_COLS


def _fmt_body(in_ref, out_ref):
    x = in_ref[...]
    for g in range(FMT_COLS // 2048):
        r5 = jnp.concatenate(
            [x[:, g * 2048 + 128 * t: g * 2048 + 128 * (t + 1)]
             for t in range(16)], axis=0)
        out_ref[pl.ds(g * 128, 128), :] = r5.T


_fmt = pl.pallas_call(
    _fmt_body,
    grid=(FMT_GRID,),
    in_specs=[pl.BlockSpec((D, FMT_COLS), lambda k: (0, k))],
    out_specs=pl.BlockSpec((FMT_COLS // 16, 128), lambda k: (k, 0)),
    out_shape=jax.ShapeDtypeStruct((VP // 16, 128), jnp.float32),
)


XP_BL = 16


def _xprep_body(in_ref, out_ref):
    xi = in_ref[...]
    xm = ((xi & ~jnp.int32(2047)) | ((xi & 127) << 4) | ((xi >> 7) & 15))
    for m in range(XP_BL):
        out_ref[pl.ds(m * L, L), :] = xm[:, m * CBL:(m + 1) * CBL]


_xprep = pl.pallas_call(
    _xprep_body,
    grid=(B // (CBL * XP_BL),),
    in_specs=[pl.BlockSpec((L, CBL * XP_BL), lambda k: (0, k))],
    out_specs=pl.BlockSpec((L * XP_BL, CBL), lambda k: (k, 0)),
    out_shape=jax.ShapeDtypeStruct(((B // CBL) * L, CBL), jnp.int32),
)


def _pool_body(x_hbm, emb_hbm, out_hbm, idx0, idx1, rows0, rows1, rows2,
               rows3, pooled_v, isem0, isem1, gsem0, gsem1, gsem2, gsem3):
    wid = lax.axis_index("c") * 16 + lax.axis_index("s")
    inv_l = jnp.float32(1.0 / L)

    l16 = lax.iota(jnp.int32, 16)
    col = lax.bitwise_and(l16, 7)
    half01 = lax.shift_right_logical(l16, 3)

    idx_bufs = (idx0, idx1)
    isems = (isem0, isem1)
    row_bufs = (rows0, rows1, rows2, rows3)
    gsems = (gsem0, gsem1, gsem2, gsem3)

    def idx_copy(g):
        s = g % 2
        return pltpu.make_async_copy(
            x_hbm.at[pl.ds((NCH * wid + g) * CH_IDX, CH_IDX)],
            idx_bufs[s], isems[s])

    def gather(g, ss, rs):
        return pltpu.make_async_copy(
            emb_hbm.at[idx_bufs[g % 2].at[pl.ds(ss * SR, SR)]],
            row_bufs[rs], gsems[rs])

    idx_copy(0).start()
    idx_copy(0).wait()
    gather(0, 0, 0).start()
    gather(0, 1, 1).start()
    gather(0, 2, 2).start()
    for k in range(NSTEP):
        g, ss = divmod(k, SUBS)
        if ss == 0 and g + 1 < NCH:
            idx_copy(g + 1).start()
        k2 = k + 3
        if k2 < NSTEP:
            g2, ss2 = divmod(k2, SUBS)
            if ss2 == 0:
                idx_copy(g2).wait()
            gather(g2, ss2, k2 % 4).start()
        gather(g, ss, k % 4).wait()
        rows_v = row_bufs[k % 4]

        def pair_body(p, carry):
            base = g * (CBL * D) + p * 16
            bvec = 2 * p + half01
            if ss == 0:
                acc = jnp.zeros((16,), jnp.float32)
            else:
                acc = pooled_v[pl.ds(base, 16)]

            def jb(j5, acc):
                for u in range(5):
                    r = (5 * j5 + u) * CBL + bvec
                    acc = acc + plsc.load_gather(rows_v, [r, col])
                return acc

            acc = lax.fori_loop(0, SR // (5 * CBL), jb, acc)
            if ss == SUBS - 1:
                acc = acc * inv_l
            pooled_v[pl.ds(base, 16)] = acc
            return carry

        lax.fori_loop(0, CBL // 2, pair_body, 0)

    pltpu.sync_copy(pooled_v, out_hbm.at[pl.ds(wid * (ROWS_PER_W * D),
                                               ROWS_PER_W * D)])


_pool = pl.kernel(
    _pool_body,
    out_type=jax.ShapeDtypeStruct((B * D,), jnp.float32),
    mesh=plsc.VectorSubcoreMesh(core_axis_name="c", subcore_axis_name="s"),
    compiler_params=pltpu.CompilerParams(needs_layout_passes=False,
                                         use_tc_tiling_on_sc=False),
    scratch_types=[
        pltpu.VMEM((CH_IDX,), jnp.int32),
        pltpu.VMEM((CH_IDX,), jnp.int32),
        pltpu.VMEM((SR, D), jnp.float32),
        pltpu.VMEM((SR, D), jnp.float32),
        pltpu.VMEM((SR, D), jnp.float32),
        pltpu.VMEM((SR, D), jnp.float32),
        pltpu.VMEM((ROWS_PER_W * D,), jnp.float32),
        pltpu.SemaphoreType.DMA,
        pltpu.SemaphoreType.DMA,
        pltpu.SemaphoreType.DMA,
        pltpu.SemaphoreType.DMA,
        pltpu.SemaphoreType.DMA,
        pltpu.SemaphoreType.DMA,
    ],
)


def _mlp_body(h_ref, w1_ref, b1_ref, w2_ref, b2_ref, out_ref):
    h = h_ref[...]
    a = jnp.dot(h, w1_ref[...], preferred_element_type=jnp.float32)
    a = jnp.maximum(a + b1_ref[...], 0.0)
    z = jnp.sum(a * w2_ref[...][:, 0][None, :], axis=1, keepdims=True)
    z = z + b2_ref[...]
    out_ref[...] = 1.0 / (1.0 + jnp.exp(-z))


def _mlp(pooled, w1, b1, w2, b2):
    return pl.pallas_call(
        _mlp_body,
        out_shape=jax.ShapeDtypeStruct((B, 1), jnp.float32),
    )(pooled, w1, b1, w2, b2)


@jax.jit
def kernel(x, emb, W1, b1, W2, b2):
    x_flat = _xprep(x.astype(jnp.int32).T).reshape(-1)
    emb2 = _fmt(emb.T).reshape(-1).reshape(VP, D)
    pooled = _pool(x_flat, emb2).reshape(B, D)
    return _mlp(pooled, W1, b1, W2, b2)


# speedup vs baseline: 1.1910x; 1.1910x over previous
import jax
import jax.numpy as jnp
from jax import lax
from jax.experimental import pallas as pl
from jax.experimental.pallas import tpu as pltpu
from jax.experimental.pallas import tpu_sc as plsc

B = 16384
L = 200
D = 8
NW = 32
ROWS_PER_W = B // NW
CBL = 128
NCH = ROWS_PER_W // CBL
CH_IDX = CBL * L
SUBS = 10
SR = CH_IDX // SUBS
NSTEP = NCH * SUBS

FMT_GRID = 16
FMT_COLS = 65536
VP = FMT_GRID * FMT_COLS


def _fmt_body(in_ref, out_ref):
    x = in_ref[...]
    for g in range(FMT_COLS // 2048):
        r5 = jnp.concatenate(
            [x[:, g * 2048 + 128 * t: g * 2048 + 128 * (t + 1)]
             for t in range(16)], axis=0)
        out_ref[pl.ds(g * 128, 128), :] = r5.T


_fmt = pl.pallas_call(
    _fmt_body,
    grid=(FMT_GRID,),
    in_specs=[pl.BlockSpec((D, FMT_COLS), lambda k: (0, k))],
    out_specs=pl.BlockSpec((FMT_COLS // 16, 128), lambda k: (k, 0)),
    out_shape=jax.ShapeDtypeStruct((VP // 16, 128), jnp.float32),
)


XP_BL = 16


def _xprep_body(in_ref, out_ref):
    xi = in_ref[...]
    xm = ((xi & ~jnp.int32(2047)) | ((xi & 127) << 4) | ((xi >> 7) & 15))
    for m in range(XP_BL):
        out_ref[pl.ds(m * L, L), :] = xm[:, m * CBL:(m + 1) * CBL]


_xprep = pl.pallas_call(
    _xprep_body,
    grid=(B // (CBL * XP_BL),),
    in_specs=[pl.BlockSpec((L, CBL * XP_BL), lambda k: (0, k))],
    out_specs=pl.BlockSpec((L * XP_BL, CBL), lambda k: (k, 0)),
    out_shape=jax.ShapeDtypeStruct(((B // CBL) * L, CBL), jnp.int32),
)


def _pool_body(x_hbm, emb_hbm, out_hbm, idx0, idx1, rows0, rows1, rows2,
               pooled_v, isem0, isem1, gsem0, gsem1, gsem2):
    wid = lax.axis_index("c") * 16 + lax.axis_index("s")
    inv_l = jnp.float32(1.0 / L)

    l16 = lax.iota(jnp.int32, 16)
    col = lax.bitwise_and(l16, 7)
    half01 = lax.shift_right_logical(l16, 3)

    idx_bufs = (idx0, idx1)
    isems = (isem0, isem1)
    row_bufs = (rows0, rows1, rows2)
    gsems = (gsem0, gsem1, gsem2)

    def idx_copy(g):
        s = g % 2
        return pltpu.make_async_copy(
            x_hbm.at[pl.ds((NCH * wid + g) * CH_IDX, CH_IDX)],
            idx_bufs[s], isems[s])

    def gather(g, ss, rs):
        return pltpu.make_async_copy(
            emb_hbm.at[idx_bufs[g % 2].at[pl.ds(ss * SR, SR)]],
            row_bufs[rs], gsems[rs])

    idx_copy(0).start()
    idx_copy(0).wait()
    gather(0, 0, 0).start()
    gather(0, 1, 1).start()
    for k in range(NSTEP):
        g, ss = divmod(k, SUBS)
        if ss == 0 and g + 1 < NCH:
            idx_copy(g + 1).start()
        k2 = k + 2
        if k2 < NSTEP:
            g2, ss2 = divmod(k2, SUBS)
            if ss2 == 0:
                idx_copy(g2).wait()
            gather(g2, ss2, k2 % 3).start()
        gather(g, ss, k % 3).wait()
        rows_v = row_bufs[k % 3]

        def pair_body(p, carry):
            base = g * (CBL * D) + p * 16
            bvec = 2 * p + half01
            if ss == 0:
                acc = jnp.zeros((16,), jnp.float32)
            else:
                acc = pooled_v[pl.ds(base, 16)]

            def jb(j5, acc):
                for u in range(5):
                    r = (5 * j5 + u) * CBL + bvec
                    acc = acc + plsc.load_gather(rows_v, [r, col])
                return acc

            acc = lax.fori_loop(0, SR // (5 * CBL), jb, acc)
            if ss == SUBS - 1:
                acc = acc * inv_l
            pooled_v[pl.ds(base, 16)] = acc
            return carry

        lax.fori_loop(0, CBL // 2, pair_body, 0)

    pltpu.sync_copy(pooled_v, out_hbm.at[pl.ds(wid * (ROWS_PER_W * D),
                                               ROWS_PER_W * D)])


_pool = pl.kernel(
    _pool_body,
    out_type=jax.ShapeDtypeStruct((B * D,), jnp.float32),
    mesh=plsc.VectorSubcoreMesh(core_axis_name="c", subcore_axis_name="s"),
    compiler_params=pltpu.CompilerParams(needs_layout_passes=False,
                                         use_tc_tiling_on_sc=False),
    scratch_types=[
        pltpu.VMEM((CH_IDX,), jnp.int32),
        pltpu.VMEM((CH_IDX,), jnp.int32),
        pltpu.VMEM((SR, D), jnp.float32),
        pltpu.VMEM((SR, D), jnp.float32),
        pltpu.VMEM((SR, D), jnp.float32),
        pltpu.VMEM((ROWS_PER_W * D,), jnp.float32),
        pltpu.SemaphoreType.DMA,
        pltpu.SemaphoreType.DMA,
        pltpu.SemaphoreType.DMA,
        pltpu.SemaphoreType.DMA,
        pltpu.SemaphoreType.DMA,
    ],
)


def _mlp_body(h_ref, w1_ref, b1_ref, w2_ref, b2_ref, out_ref):
    h = h_ref[...]
    a = jnp.dot(h, w1_ref[...], preferred_element_type=jnp.float32)
    a = jnp.maximum(a + b1_ref[...], 0.0)
    z = jnp.sum(a * w2_ref[...][:, 0][None, :], axis=1, keepdims=True)
    z = z + b2_ref[...]
    out_ref[...] = 1.0 / (1.0 + jnp.exp(-z))


def _mlp(pooled, w1, b1, w2, b2):
    return pl.pallas_call(
        _mlp_body,
        out_shape=jax.ShapeDtypeStruct((B, 1), jnp.float32),
    )(pooled, w1, b1, w2, b2)


@jax.jit
def kernel(x, emb, W1, b1, W2, b2):
    x_flat = _xprep(x.astype(jnp.int32).T).reshape(-1)
    emb2 = _fmt(emb.T).reshape(-1).reshape(VP, D)
    pooled = _pool(x_flat, emb2).reshape(B, D)
    return _mlp(pooled, W1, b1, W2, b2)


# speedup vs baseline: 1.3012x; 1.0925x over previous
import jax
import jax.numpy as jnp
from jax import lax
from jax.experimental import pallas as pl
from jax.experimental.pallas import tpu as pltpu
from jax.experimental.pallas import tpu_sc as plsc

B = 16384
L = 200
D = 8
NW = 32
ROWS_PER_W = B // NW
CBL = 128
NCH = ROWS_PER_W // CBL
CH_IDX = CBL * L
SUBS = 10
SR = CH_IDX // SUBS
NSTEP = NCH * SUBS

FMT_GRID = 16
FMT_COLS = 65536
VP = FMT_GRID * FMT_COLS


def _fmt_body(in_ref, out_ref):
    x = in_ref[...]
    for g in range(FMT_COLS // 2048):
        r5 = jnp.concatenate(
            [x[:, g * 2048 + 128 * t: g * 2048 + 128 * (t + 1)]
             for t in range(16)], axis=0)
        out_ref[pl.ds(g * 128, 128), :] = r5.T


_fmt = pl.pallas_call(
    _fmt_body,
    grid=(FMT_GRID,),
    in_specs=[pl.BlockSpec((D, FMT_COLS), lambda k: (0, k))],
    out_specs=pl.BlockSpec((FMT_COLS // 16, 128), lambda k: (k, 0)),
    out_shape=jax.ShapeDtypeStruct((VP // 16, 128), jnp.float32),
)


XP_BL = 16


def _xprep_body(in_ref, out_ref):
    xi = in_ref[...]
    xm = ((xi & ~jnp.int32(2047)) | ((xi & 127) << 4) | ((xi >> 7) & 15))
    for m in range(XP_BL):
        out_ref[pl.ds(m * L, L), :] = xm[:, m * CBL:(m + 1) * CBL]


_xprep = pl.pallas_call(
    _xprep_body,
    grid=(B // (CBL * XP_BL),),
    in_specs=[pl.BlockSpec((L, CBL * XP_BL), lambda k: (0, k))],
    out_specs=pl.BlockSpec((L * XP_BL, CBL), lambda k: (k, 0)),
    out_shape=jax.ShapeDtypeStruct(((B // CBL) * L, CBL), jnp.int32),
)


def _pool_body(x_hbm, emb_hbm, out_hbm, idx0, idx1, rows0, rows1, rows2,
               pooled_v, isem0, isem1, gsem0, gsem1, gsem2):
    wid = lax.axis_index("c") * 16 + lax.axis_index("s")
    inv_l = jnp.float32(1.0 / L)

    l16 = lax.iota(jnp.int32, 16)
    col = lax.bitwise_and(l16, 7)
    half01 = lax.shift_right_logical(l16, 3)

    idx_bufs = (idx0, idx1)
    isems = (isem0, isem1)
    row_bufs = (rows0, rows1, rows2)
    gsems = (gsem0, gsem1, gsem2)

    def idx_copy(g):
        s = g % 2
        return pltpu.make_async_copy(
            x_hbm.at[pl.ds((NCH * wid + g) * CH_IDX, CH_IDX)],
            idx_bufs[s], isems[s])

    def gather(g, ss, rs):
        return pltpu.make_async_copy(
            emb_hbm.at[idx_bufs[g % 2].at[pl.ds(ss * SR, SR)]],
            row_bufs[rs], gsems[rs])

    idx_copy(0).start()
    idx_copy(0).wait()
    gather(0, 0, 0).start()
    gather(0, 1, 1).start()
    for k in range(NSTEP):
        g, ss = divmod(k, SUBS)
        if ss == 0 and g + 1 < NCH:
            idx_copy(g + 1).start()
        k2 = k + 2
        if k2 < NSTEP:
            g2, ss2 = divmod(k2, SUBS)
            if ss2 == 0:
                idx_copy(g2).wait()
            gather(g2, ss2, k2 % 3).start()
        gather(g, ss, k % 3).wait()
        rows_v = row_bufs[k % 3]

        def pair_body(p, carry):
            base = g * (CBL * D) + p * 16
            bvec = 2 * p + half01
            if ss == 0:
                acc = jnp.zeros((16,), jnp.float32)
            else:
                acc = pooled_v[pl.ds(base, 16)]

            def jb(j5, acc):
                for u in range(5):
                    r = (5 * j5 + u) * CBL + bvec
                    acc = acc + plsc.load_gather(rows_v, [r, col])
                return acc

            acc = lax.fori_loop(0, SR // (5 * CBL), jb, acc)
            if ss == SUBS - 1:
                acc = acc * inv_l
            pooled_v[pl.ds(base, 16)] = acc
            return carry

        lax.fori_loop(0, CBL // 2, pair_body, 0)

    pltpu.sync_copy(pooled_v, out_hbm.at[pl.ds(wid * (ROWS_PER_W * D),
                                               ROWS_PER_W * D)])


_pool = pl.kernel(
    _pool_body,
    out_type=jax.ShapeDtypeStruct((B * D,), jnp.float32),
    mesh=plsc.VectorSubcoreMesh(core_axis_name="c", subcore_axis_name="s"),
    compiler_params=pltpu.CompilerParams(needs_layout_passes=False,
                                         use_tc_tiling_on_sc=False),
    scratch_types=[
        pltpu.VMEM((CH_IDX,), jnp.int32),
        pltpu.VMEM((CH_IDX,), jnp.int32),
        pltpu.VMEM((SR, D), jnp.float32),
        pltpu.VMEM((SR, D), jnp.float32),
        pltpu.VMEM((SR, D), jnp.float32),
        pltpu.VMEM((ROWS_PER_W * D,), jnp.float32),
        pltpu.SemaphoreType.DMA,
        pltpu.SemaphoreType.DMA,
        pltpu.SemaphoreType.DMA,
        pltpu.SemaphoreType.DMA,
        pltpu.SemaphoreType.DMA,
    ],
)


def _mlp_body(h_ref, w1_ref, b1_ref, w2_ref, b2_ref, out_ref):
    h = h_ref[...]
    a = jnp.dot(h, w1_ref[...], preferred_element_type=jnp.float32)
    a = jnp.maximum(a + b1_ref[...], 0.0)
    z = jnp.dot(a, w2_ref[...], preferred_element_type=jnp.float32)
    z = z + b2_ref[...]
    out_ref[...] = 1.0 / (1.0 + jnp.exp(-z))


def _mlp(pooled16, w1r, b1r, w2r, b2):
    return pl.pallas_call(
        _mlp_body,
        out_shape=jax.ShapeDtypeStruct((B // 16, 16), jnp.float32),
    )(pooled16, w1r, b1r, w2r, b2)


@jax.jit
def kernel(x, emb, W1, b1, W2, b2):
    x_flat = _xprep(x.astype(jnp.int32).T).reshape(-1)
    emb2 = _fmt(emb.T).reshape(-1).reshape(VP, D)
    pooled16 = _pool(x_flat, emb2).reshape(B // 16, 16 * D)
    eye16 = jnp.eye(16, dtype=jnp.float32)
    w1r = jnp.kron(eye16, W1)
    b1r = jnp.tile(b1, 16)
    w2r = jnp.kron(eye16, W2)
    return _mlp(pooled16, w1r, b1r, w2r, b2).reshape(B, 1)


# speedup vs baseline: 1.3288x; 1.0212x over previous
import jax
import jax.numpy as jnp
from jax import lax
from jax.experimental import pallas as pl
from jax.experimental.pallas import tpu as pltpu
from jax.experimental.pallas import tpu_sc as plsc

B = 16384
L = 200
D = 8
NW = 32
ROWS_PER_W = B // NW
CBL = 128
NCH = ROWS_PER_W // CBL
CH_IDX = CBL * L
SUBS = 10
SR = CH_IDX // SUBS
NSTEP = NCH * SUBS

FMT_GRID = 16
FMT_COLS = 65536
VP = FMT_GRID * FMT_COLS


XP_BL = (B // CBL) // FMT_GRID


def _prep_body(emb_ref, x_ref, tab_ref, idx_ref):
    x = emb_ref[...]
    for g in range(FMT_COLS // 2048):
        r5 = jnp.concatenate(
            [x[:, g * 2048 + 128 * t: g * 2048 + 128 * (t + 1)]
             for t in range(16)], axis=0)
        tab_ref[pl.ds(g * 128, 128), :] = r5.T
    xi = x_ref[...]
    xm = ((xi & ~jnp.int32(2047)) | ((xi & 127) << 4) | ((xi >> 7) & 15))
    for m in range(XP_BL):
        idx_ref[pl.ds(m * L, L), :] = xm[:, m * CBL:(m + 1) * CBL]


_prep = pl.pallas_call(
    _prep_body,
    grid=(FMT_GRID,),
    in_specs=[pl.BlockSpec((D, FMT_COLS), lambda k: (0, k)),
              pl.BlockSpec((L, CBL * XP_BL), lambda k: (0, k))],
    out_specs=[pl.BlockSpec((FMT_COLS // 16, 128), lambda k: (k, 0)),
               pl.BlockSpec((L * XP_BL, CBL), lambda k: (k, 0))],
    out_shape=[jax.ShapeDtypeStruct((VP // 16, 128), jnp.float32),
               jax.ShapeDtypeStruct(((B // CBL) * L, CBL), jnp.int32)],
)


def _pool_body(x_hbm, emb_hbm, out_hbm, idx0, idx1, rows0, rows1, rows2,
               pooled_v, isem0, isem1, gsem0, gsem1, gsem2):
    wid = lax.axis_index("c") * 16 + lax.axis_index("s")
    inv_l = jnp.float32(1.0 / L)

    l16 = lax.iota(jnp.int32, 16)
    col = lax.bitwise_and(l16, 7)
    half01 = lax.shift_right_logical(l16, 3)

    idx_bufs = (idx0, idx1)
    isems = (isem0, isem1)
    row_bufs = (rows0, rows1, rows2)
    gsems = (gsem0, gsem1, gsem2)

    def idx_copy(g):
        s = g % 2
        return pltpu.make_async_copy(
            x_hbm.at[pl.ds((NCH * wid + g) * CH_IDX, CH_IDX)],
            idx_bufs[s], isems[s])

    def gather(g, ss, rs):
        return pltpu.make_async_copy(
            emb_hbm.at[idx_bufs[g % 2].at[pl.ds(ss * SR, SR)]],
            row_bufs[rs], gsems[rs])

    idx_copy(0).start()
    idx_copy(0).wait()
    gather(0, 0, 0).start()
    gather(0, 1, 1).start()
    for k in range(NSTEP):
        g, ss = divmod(k, SUBS)
        if ss == 0 and g + 1 < NCH:
            idx_copy(g + 1).start()
        k2 = k + 2
        if k2 < NSTEP:
            g2, ss2 = divmod(k2, SUBS)
            if ss2 == 0:
                idx_copy(g2).wait()
            gather(g2, ss2, k2 % 3).start()
        gather(g, ss, k % 3).wait()
        rows_v = row_bufs[k % 3]

        def pair_body(p, carry):
            base = g * (CBL * D) + p * 16
            bvec = 2 * p + half01
            if ss == 0:
                acc = jnp.zeros((16,), jnp.float32)
            else:
                acc = pooled_v[pl.ds(base, 16)]

            def jb(j5, acc):
                for u in range(5):
                    r = (5 * j5 + u) * CBL + bvec
                    acc = acc + plsc.load_gather(rows_v, [r, col])
                return acc

            acc = lax.fori_loop(0, SR // (5 * CBL), jb, acc)
            if ss == SUBS - 1:
                acc = acc * inv_l
            pooled_v[pl.ds(base, 16)] = acc
            return carry

        lax.fori_loop(0, CBL // 2, pair_body, 0)

    pltpu.sync_copy(pooled_v, out_hbm.at[pl.ds(wid * (ROWS_PER_W * D),
                                               ROWS_PER_W * D)])


_pool = pl.kernel(
    _pool_body,
    out_type=jax.ShapeDtypeStruct((B * D,), jnp.float32),
    mesh=plsc.VectorSubcoreMesh(core_axis_name="c", subcore_axis_name="s"),
    compiler_params=pltpu.CompilerParams(needs_layout_passes=False,
                                         use_tc_tiling_on_sc=False),
    scratch_types=[
        pltpu.VMEM((CH_IDX,), jnp.int32),
        pltpu.VMEM((CH_IDX,), jnp.int32),
        pltpu.VMEM((SR, D), jnp.float32),
        pltpu.VMEM((SR, D), jnp.float32),
        pltpu.VMEM((SR, D), jnp.float32),
        pltpu.VMEM((ROWS_PER_W * D,), jnp.float32),
        pltpu.SemaphoreType.DMA,
        pltpu.SemaphoreType.DMA,
        pltpu.SemaphoreType.DMA,
        pltpu.SemaphoreType.DMA,
        pltpu.SemaphoreType.DMA,
    ],
)


def _mlp_body(h_ref, w1_ref, b1_ref, w2_ref, b2_ref, out_ref):
    h = h_ref[...]
    a = jnp.dot(h, w1_ref[...], preferred_element_type=jnp.float32)
    a = jnp.maximum(a + b1_ref[...], 0.0)
    z = jnp.dot(a, w2_ref[...], preferred_element_type=jnp.float32)
    z = z + b2_ref[...]
    out_ref[...] = 1.0 / (1.0 + jnp.exp(-z))


def _mlp(pooled16, w1r, b1r, w2r, b2):
    return pl.pallas_call(
        _mlp_body,
        out_shape=jax.ShapeDtypeStruct((B // 16, 16), jnp.float32),
    )(pooled16, w1r, b1r, w2r, b2)


@jax.jit
def kernel(x, emb, W1, b1, W2, b2):
    table, idxs = _prep(emb.T, x.astype(jnp.int32).T)
    x_flat = idxs.reshape(-1)
    emb2 = table.reshape(-1).reshape(VP, D)
    pooled16 = _pool(x_flat, emb2).reshape(B // 16, 16 * D)
    eye16 = jnp.eye(16, dtype=jnp.float32)
    w1r = jnp.kron(eye16, W1)
    b1r = jnp.tile(b1, 16)
    w2r = jnp.kron(eye16, W2)
    return _mlp(pooled16, w1r, b1r, w2r, b2).reshape(B, 1)
